# baseline (speedup 1.0000x reference)
import jax
import jax.numpy as jnp
from jax.experimental import pallas as pl


def _gcn_body(x_ref, adj_ref, W1_ref, b1_ref, W2_ref, b2_ref, out_ref):
    adj = adj_ref[...]
    deg = jnp.sum(adj, axis=0)
    dinv = jnp.where(deg > 0.0, jax.lax.rsqrt(jnp.where(deg > 0.0, deg, 1.0)), 0.0)
    drow = dinv[None, :]

    xwT = jax.lax.dot_general(
        W1_ref[...], x_ref[...], (((0,), (1,)), ((), ())),
        preferred_element_type=jnp.float32,
    )
    t1 = jnp.dot(xwT * drow, adj, preferred_element_type=jnp.float32)
    hT = jnp.maximum(t1 * drow + b1_ref[...], 0.0)

    hwT = jax.lax.dot_general(
        W2_ref[...], hT, (((0,), (0,)), ((), ())),
        preferred_element_type=jnp.float32,
    )
    t2 = jnp.dot(hwT * drow, adj, preferred_element_type=jnp.float32)
    out_ref[...] = t2 * drow + b2_ref[...]


def kernel(x, adj, W1, b1, W2, b2):
    n = x.shape[0]
    outT = pl.pallas_call(
        _gcn_body,
        out_shape=jax.ShapeDtypeStruct((W2.shape[1], n), x.dtype),
    )(x, adj, W1, b1.reshape(-1, 1), W2, b2.reshape(-1, 1))
    return outT.T


# speedup vs baseline: 1.3805x; 1.3805x over previous
import jax
import jax.numpy as jnp
from jax.experimental import pallas as pl
from jax.experimental.pallas import tpu as pltpu

_N = 1024
_HALF = _N // 2


def _gcn_body(x_ref, adj_hbm, W1_ref, b1_ref, W2_ref, b2_ref, out_ref,
              adj_vmem, sems):
    copies = [
        pltpu.make_async_copy(
            adj_hbm.at[pl.ds(c * _HALF, _HALF), :],
            adj_vmem.at[pl.ds(c * _HALF, _HALF), :],
            sems.at[c],
        )
        for c in range(2)
    ]
    for cp in copies:
        cp.start()

    xw = jnp.dot(x_ref[...], W1_ref[...], preferred_element_type=jnp.float32)

    copies[0].wait()
    deg0 = jnp.sum(adj_vmem[pl.ds(0, _HALF), :], axis=0)
    copies[1].wait()
    deg = deg0 + jnp.sum(adj_vmem[pl.ds(_HALF, _HALF), :], axis=0)

    dinv = jnp.where(deg > 0.0, jax.lax.rsqrt(jnp.where(deg > 0.0, deg, 1.0)), 0.0)
    dcol = dinv[:, None]

    adj = adj_vmem[...]
    dn = (((0,), (0,)), ((), ()))
    t1 = jax.lax.dot_general(adj, xw * dcol, dn, preferred_element_type=jnp.float32)
    h = jnp.maximum(t1 * dcol + b1_ref[...], 0.0)

    hw = jnp.dot(h, W2_ref[...], preferred_element_type=jnp.float32)
    t2 = jax.lax.dot_general(adj, hw * dcol, dn, preferred_element_type=jnp.float32)
    out_ref[...] = t2 * dcol + b2_ref[...]


def kernel(x, adj, W1, b1, W2, b2):
    n = x.shape[0]
    return pl.pallas_call(
        _gcn_body,
        out_shape=jax.ShapeDtypeStruct((n, W2.shape[1]), x.dtype),
        in_specs=[
            pl.BlockSpec(memory_space=pltpu.MemorySpace.VMEM),
            pl.BlockSpec(memory_space=pltpu.MemorySpace.HBM),
            pl.BlockSpec(memory_space=pltpu.MemorySpace.VMEM),
            pl.BlockSpec(memory_space=pltpu.MemorySpace.VMEM),
            pl.BlockSpec(memory_space=pltpu.MemorySpace.VMEM),
            pl.BlockSpec(memory_space=pltpu.MemorySpace.VMEM),
        ],
        scratch_shapes=[
            pltpu.VMEM((_N, _N), jnp.float32),
            pltpu.SemaphoreType.DMA((2,)),
        ],
    )(x, adj, W1, b1.reshape(1, -1), W2, b2.reshape(1, -1))


# speedup vs baseline: 1.5856x; 1.1485x over previous
import jax
import jax.numpy as jnp
from jax.experimental import pallas as pl


def _gcn_body(x_ref, adj_ref, W1_ref, b1_ref, W2_ref, b2_ref, out_ref):
    adj = adj_ref[...]
    deg = jnp.sum(adj, axis=0)
    dinv = jnp.where(deg > 0.0, jax.lax.rsqrt(jnp.where(deg > 0.0, deg, 1.0)), 0.0)
    dcol = dinv[:, None]

    xw = jnp.dot(x_ref[...], W1_ref[...], preferred_element_type=jnp.float32)
    t1 = jax.lax.dot_general(
        adj, xw * dcol, (((0,), (0,)), ((), ())), preferred_element_type=jnp.float32
    )
    h = jnp.maximum(t1 * dcol + b1_ref[...], 0.0)

    hw = jnp.dot(h, W2_ref[...], preferred_element_type=jnp.float32)
    t2 = jax.lax.dot_general(
        adj, hw * dcol, (((0,), (0,)), ((), ())), preferred_element_type=jnp.float32
    )
    out_ref[...] = t2 * dcol + b2_ref[...]


def kernel(x, adj, W1, b1, W2, b2):
    n = x.shape[0]
    return pl.pallas_call(
        _gcn_body,
        out_shape=jax.ShapeDtypeStruct((n, W2.shape[1]), x.dtype),
    )(x, adj, W1, b1.reshape(1, -1), W2, b2.reshape(1, -1))
